# baseline (speedup 1.0000x reference)
import functools

import jax
import jax.numpy as jnp
from jax import lax
from jax.experimental import pallas as pl
from jax.experimental.pallas import tpu as pltpu
from jax.experimental.pallas import tpu_sc as plsc

F = 26
V = 100000
D = 32
B = 4096

NC = 2
NS = 16
NW = NC * NS

CW = 1664
NCH = 61
SW62 = 128
TAIL = 128
TOFF = (NCH - 1) * CW + SW62 - (V - TAIL)
SUPW = 8 * CW
LCAP = 4112

BPW = B // NW
RPW = BPW * F
CHUNK = 128
NCHUNK = RPW // CHUNK
GROUPS = BPW // 16

_mesh = plsc.VectorSubcoreMesh(core_axis_name="c", subcore_axis_name="s")


@functools.partial(
    pl.kernel,
    mesh=_mesh,
    compiler_params=pltpu.CompilerParams(
        needs_layout_passes=False, use_tc_tiling_on_sc=True),
    out_type=[
        jax.ShapeDtypeStruct((F * 64 * LCAP,), jnp.int32),
        jax.ShapeDtypeStruct((F * 64,), jnp.int32),
    ],
    scratch_types=[
        pltpu.VMEM((32, 128), jnp.int32),
        pltpu.VMEM((8 * LCAP,), jnp.int32),
        pltpu.VMEM((8 * LCAP,), jnp.int32),
        pltpu.VMEM((64,), jnp.int32),
        pltpu.SemaphoreType.DMA,
        pltpu.SemaphoreType.DMA,
    ],
)
def _bucket_kernel(idx3, lists_out, cnt_out, idxf, supl, subl, bcnt_v,
                   isem, osem):
    w = lax.axis_index("s") * NC + lax.axis_index("c")
    iota = lax.iota(jnp.int32, 16)
    zeros_i = jnp.zeros((16,), jnp.int32)

    @pl.when(w < F)
    def _():
        f = w
        pltpu.async_copy(idx3.at[f], idxf, isem).wait()
        for q in range(4):
            bcnt_v[pl.ds(q * 16, 16)] = zeros_i

        def _p1(gi, cnt):
            r = gi // 8
            k = gi - r * 8
            v16 = idxf[r, pl.ds(k * 16, 16)]
            b16 = gi * 16 + iota
            for s in range(8):
                vloc = v16 - s * SUPW
                m = (vloc >= 0) & (vloc < SUPW)
                plsc.store_compressed(
                    supl.at[pl.ds(s * LCAP + cnt[s], 16)],
                    vloc * 4096 + b16, mask=m)
                pop = plsc.all_reduce_population_count(m)
                cnt = cnt + jnp.where(iota == s, pop, 0)
            return cnt

        cnt1 = lax.fori_loop(0, 256, _p1, zeros_i)

        ocopies = []
        for s in range(8):
            ns = cnt1[s]
            nsub = 8

            def _p2(gi, cnt, s=s, ns=ns, nsub=nsub):
                li = gi * 16 + iota
                lim = jnp.minimum(li, ns - 1)
                valid = li < ns
                p = plsc.load_gather(supl, [s * LCAP + lim])
                vl = p // 4096
                b = p - vl * 4096
                for t in range(nsub):
                    vs = vl - t * CW
                    m = valid & (vs >= 0) & (vs < CW)
                    plsc.store_compressed(
                        subl.at[pl.ds(t * LCAP + cnt[t], 16)],
                        vs * 4096 + b, mask=m)
                    pop = plsc.all_reduce_population_count(m)
                    cnt = cnt + jnp.where(iota == t, pop, 0)
                return cnt

            cnt2 = lax.fori_loop(0, (ns + 15) // 16, _p2, zeros_i)
            plsc.store_scatter(bcnt_v, [s * 8 + iota], cnt2,
                               mask=iota < nsub)
            ocopies.append(pltpu.async_copy(
                subl, lists_out.at[pl.ds((f * 64 + s * 8) * LCAP, 8 * LCAP)],
                osem))
            ocopies[-1].wait()

        pltpu.sync_copy(bcnt_v, cnt_out.at[pl.ds(f * 64, 64)])


@functools.partial(
    pl.kernel,
    mesh=_mesh,
    compiler_params=pltpu.CompilerParams(
        needs_layout_passes=False, use_tc_tiling_on_sc=True),
    out_type=jax.ShapeDtypeStruct((B * F, 128), jnp.float32),
    scratch_types=[
        pltpu.VMEM((4, 8, CW), jnp.float32),
        pltpu.VMEM((4, 8, CW), jnp.float32),
        pltpu.VMEM((4, 8, TAIL), jnp.float32),
        pltpu.VMEM((LCAP,), jnp.int32),
        pltpu.VMEM((64,), jnp.int32),
        pltpu.VMEM((16, 128), jnp.float32),
        pltpu.VMEM((16, 128), jnp.float32),
        pltpu.VMEM((16, 128), jnp.float32),
        pltpu.VMEM((16, 128), jnp.float32),
        pltpu.VMEM((16,), jnp.int32),
        pltpu.VMEM((16,), jnp.int32),
        pltpu.VMEM((16,), jnp.int32),
        pltpu.VMEM((16,), jnp.int32),
        pltpu.SemaphoreType.DMA,
        pltpu.SemaphoreType.DMA,
        pltpu.SemaphoreType.DMA,
        pltpu.SemaphoreType.DMA,
        pltpu.SemaphoreType.DMA,
        pltpu.SemaphoreType.DMA,
        pltpu.SemaphoreType.DMA,
        pltpu.SemaphoreType.DMA,
        pltpu.SemaphoreType.DMA,
    ],
)
def _gather_kernel(embn, emb_tail, lists_in, cnt_in, rows_out,
                   slab_a, slab_b, tslab, list_v, cnt_v,
                   stage0, stage1, stage2, stage3,
                   ridx0, ridx1, ridx2, ridx3,
                   sem_a, sem_b, tsem, csem, lsem,
                   ssem0, ssem1, ssem2, ssem3):
    w = lax.axis_index("s") * NC + lax.axis_index("c")
    c0 = w
    c1 = w + 32
    iota = lax.iota(jnp.int32, 16)

    NRING = 4
    stages = (stage0, stage1, stage2, stage3)
    ridxs = (ridx0, ridx1, ridx2, ridx3)
    ssems = (ssem0, ssem1, ssem2, ssem3)

    def slab_issue(c, f, buf, sem):
        @pl.when(c < NCH - 1)
        def _():
            pltpu.async_copy(
                embn.at[pl.ds(f * 4, 4), :,
                        pl.ds(pl.multiple_of(c * CW, 128), CW)], buf, sem)

        @pl.when(c == NCH - 1)
        def _():
            pltpu.async_copy(
                embn.at[pl.ds(f * 4, 4), :, pl.ds((NCH - 1) * CW, SW62)],
                buf.at[:, :, pl.ds(0, SW62)], sem)

    def slab_drain(c, buf, sem):
        @pl.when(c < NCH - 1)
        def _():
            pltpu.make_async_copy(
                embn.at[pl.ds(0, 4), :, pl.ds(0, CW)], buf, sem).wait()

        @pl.when(c == NCH - 1)
        def _():
            pltpu.make_async_copy(
                embn.at[pl.ds(0, 4), :, pl.ds(0, SW62)],
                buf.at[:, :, pl.ds(0, SW62)], sem).wait()

    def getn(c):
        grp = c // 16
        vec = cnt_v[pl.ds(pl.multiple_of(grp * 16, 16), 16)]
        return jnp.sum(jnp.where(iota == c - grp * 16, vec, 0))

    def serve(c, n, f, slab, flags):
        sw = jnp.where(c == NCH - 1, SW62, CW)
        base = (f * 64 + c) * LCAP
        pltpu.async_copy(
            lists_in.at[pl.ds(base, 512)], list_v.at[pl.ds(0, 512)],
            lsem).wait()

        @pl.when(n > 512)
        def _():
            pltpu.async_copy(
                lists_in.at[pl.ds(base + 512, 1536)],
                list_v.at[pl.ds(512, 1536)], lsem).wait()

        @pl.when(n > 2048)
        def _():
            pltpu.async_copy(
                lists_in.at[pl.ds(base + 2048, 2048)],
                list_v.at[pl.ds(2048, 2048)], lsem).wait()

        ng = (n + 15) // 16

        def _serve_wave(t, flags):
            newflags = []
            for s in range(NRING):
                gi = t * NRING + s
                stage, ridx, ssem = stages[s], ridxs[s], ssems[s]
                fired = flags[s]

                @pl.when(gi < ng)
                def _():
                    @pl.when(fired > 0)
                    def _():
                        pltpu.make_async_copy(
                            stage, rows_out.at[ridx], ssem).wait()

                    li = jnp.minimum(gi * 16 + iota, n - 1)
                    p16 = plsc.load_gather(list_v, [li])
                    vloc = p16 // 4096
                    b16 = p16 - vloc * 4096
                    in_main = vloc < sw
                    vmain = jnp.minimum(vloc, sw - 1)
                    vtail = jnp.clip(vloc - sw + TOFF, 0, TAIL - 1)
                    for dt in range(4):
                        dtv = jnp.full((16,), dt, jnp.int32)
                        for ds in range(8):
                            dsv = jnp.full((16,), ds, jnp.int32)
                            e16 = plsc.load_gather(slab, [dtv, dsv, vmain])
                            t16 = plsc.load_gather(tslab, [dtv, dsv, vtail])
                            e16 = jnp.where(in_main, e16, t16)
                            plsc.store_scatter(
                                stage,
                                [iota,
                                 jnp.full((16,), dt * 8 + ds, jnp.int32)],
                                e16)
                    ridx[...] = b16 * F + f
                    pltpu.async_copy(stage, rows_out.at[ridx], ssem)

                newflags.append(jnp.where(gi < ng, jnp.int32(1), fired))
            return tuple(newflags)

        return lax.fori_loop(0, (ng + NRING - 1) // NRING, _serve_wave,
                             flags)

    slab_issue(c0, 0, slab_a, sem_a)

    def _field(f, flags):
        pltpu.async_copy(cnt_in.at[pl.ds(f * 64, 64)], cnt_v, csem).wait()
        n0 = getn(c0)
        n1 = getn(c1)
        slab_issue(c1, f, slab_b, sem_b)

        @pl.when(c1 == NCH - 1)
        def _():
            pltpu.async_copy(emb_tail.at[pl.ds(f * 4, 4)], tslab,
                             tsem).wait()

        slab_drain(c0, slab_a, sem_a)
        flags = serve(c0, n0, f, slab_a, flags)

        @pl.when(f < F - 1)
        def _():
            slab_issue(c0, f + 1, slab_a, sem_a)

        slab_drain(c1, slab_b, sem_b)
        flags = serve(c1, n1, f, slab_b, flags)
        return flags

    zero = jnp.int32(0)
    flags = lax.fori_loop(0, F, _field, (zero, zero, zero, zero))

    for s in range(NRING):
        @pl.when(flags[s] > 0)
        def _():
            pltpu.make_async_copy(
                stages[s], rows_out.at[ridxs[s]], ssems[s]).wait()


@functools.partial(
    pl.kernel,
    mesh=_mesh,
    compiler_params=pltpu.CompilerParams(
        needs_layout_passes=False, use_tc_tiling_on_sc=True),
    out_type=jax.ShapeDtypeStruct((B,), jnp.float32),
    scratch_types=[
        pltpu.VMEM((16 * F, 128), jnp.float32),
        pltpu.VMEM((16 * F, 128), jnp.float32),
        pltpu.VMEM((RPW,), jnp.int32),
        pltpu.VMEM((RPW,), jnp.float32),
        pltpu.VMEM((BPW,), jnp.float32),
        pltpu.SemaphoreType.DMA,
        pltpu.SemaphoreType.DMA,
        pltpu.SemaphoreType.DMA,
        pltpu.SemaphoreType.DMA,
    ],
)
def _stats_kernel(rows_in, idx_flat, bias_hbm, out1_hbm,
                  buf0, buf1, idx_v, bias_v, out_v, sem0, sem1, isem, bsem):
    w = lax.axis_index("s") * NC + lax.axis_index("c")
    base_row = w * RPW
    base_samp = w * BPW

    iota = lax.iota(jnp.int32, 16)
    zeros_f = jnp.zeros((16,), jnp.float32)
    bufs = (buf0, buf1)
    sems = (sem0, sem1)

    copies = []
    for g in range(2):
        copies.append(pltpu.async_copy(
            rows_in.at[pl.ds(base_row + g * (16 * F), 16 * F)],
            bufs[g], sems[g]))

    pltpu.async_copy(idx_flat.at[pl.ds(base_row, RPW)], idx_v, isem).wait()

    def _add_off(t, carry):
        col = t * 16
        rvec = (base_row + col) + iota
        fvec = lax.rem(rvec, F)
        idx_v[pl.ds(col, 16)] = idx_v[pl.ds(col, 16)] + fvec * V
        return carry

    lax.fori_loop(0, RPW // 16, _add_off, 0)

    bcopies = []
    for j in range(NCHUNK):
        bcopies.append(pltpu.async_copy(
            bias_hbm.at[idx_v.at[pl.ds(j * CHUNK, CHUNK)]],
            bias_v.at[pl.ds(j * CHUNK, CHUNK)], bsem))

    for g in range(GROUPS):
        s = g % 2
        buf = bufs[s]
        copies[g].wait()

        row_idx = [iota * F + f for f in range(F)]

        def _lane(d, acc):
            dvec = jnp.full((16,), d, jnp.int32)
            sa = zeros_f
            qa = zeros_f
            for f in range(F):
                e = plsc.load_gather(buf, [row_idx[f], dvec])
                sa = sa + e
                qa = qa + e * e
            return acc + (sa * sa - qa)

        acc = lax.fori_loop(0, D, _lane, zeros_f)
        out_v[pl.ds(g * 16, 16)] = 0.5 * acc

        if g + 2 < GROUPS:
            copies.append(pltpu.async_copy(
                rows_in.at[pl.ds(base_row + (g + 2) * (16 * F), 16 * F)],
                bufs[s], sems[s]))

    for cp in bcopies:
        cp.wait()

    stride = iota * F

    def _group(g, carry):
        rb = g * (16 * F)
        bias_acc = zeros_f
        for f in range(F):
            bias_acc = bias_acc + plsc.load_gather(bias_v, [stride + rb + f])
        out_v[pl.ds(g * 16, 16)] = bias_acc + out_v[pl.ds(g * 16, 16)]
        return carry

    lax.fori_loop(0, GROUPS, _group, 0)

    pltpu.sync_copy(out_v, out1_hbm.at[pl.ds(base_samp, BPW)])


def kernel(field_indices, emb_tables, bias_tables):
    idx3 = field_indices.T.reshape(F, 32, 128)
    embn = emb_tables.transpose(0, 2, 1).reshape(F * D // 8, 8, V)
    emb_tail = (emb_tables[:, V - TAIL:, :]
                .transpose(0, 2, 1).reshape(F * D // 8, 8, TAIL))
    lists, cnts = _bucket_kernel(idx3)
    rows = _gather_kernel(embn, emb_tail, lists, cnts)
    idx_flat = field_indices.reshape(B * F)
    bias_flat = bias_tables.reshape(F * V)
    out1 = _stats_kernel(rows, idx_flat, bias_flat)
    embeds = rows[:, :D].reshape(B, F, D)
    return (out1.reshape(B, 1), embeds)


# speedup vs baseline: 1.0982x; 1.0982x over previous
import functools

import jax
import jax.numpy as jnp
from jax import lax
from jax.experimental import pallas as pl
from jax.experimental.pallas import tpu as pltpu
from jax.experimental.pallas import tpu_sc as plsc

F = 26
V = 100000
D = 32
B = 4096

NC = 2
NS = 16
NW = NC * NS

CW = 1664
NCH = 61
SW62 = 128
TAIL = 128
TOFF = (NCH - 1) * CW + SW62 - (V - TAIL)
SUPW = 8 * CW
LCAP = 4112

BPW = B // NW
RPW = BPW * F
CHUNK = 128
NCHUNK = RPW // CHUNK
GROUPS = BPW // 16

_mesh = plsc.VectorSubcoreMesh(core_axis_name="c", subcore_axis_name="s")


@functools.partial(
    pl.kernel,
    mesh=_mesh,
    compiler_params=pltpu.CompilerParams(
        needs_layout_passes=False, use_tc_tiling_on_sc=True),
    out_type=[
        jax.ShapeDtypeStruct((F * 64 * LCAP,), jnp.int32),
        jax.ShapeDtypeStruct((F * 64,), jnp.int32),
    ],
    scratch_types=[
        pltpu.VMEM((32, 128), jnp.int32),
        pltpu.VMEM((8 * LCAP,), jnp.int32),
        pltpu.VMEM((8 * LCAP,), jnp.int32),
        pltpu.VMEM((64,), jnp.int32),
        pltpu.SemaphoreType.DMA,
        pltpu.SemaphoreType.DMA,
    ],
)
def _bucket_kernel(idx3, lists_out, cnt_out, idxf, supl, subl, bcnt_v,
                   isem, osem):
    w = lax.axis_index("s") * NC + lax.axis_index("c")
    iota = lax.iota(jnp.int32, 16)
    zeros_i = jnp.zeros((16,), jnp.int32)

    @pl.when(w < F)
    def _():
        f = w
        pltpu.async_copy(idx3.at[f], idxf, isem).wait()
        for q in range(4):
            bcnt_v[pl.ds(q * 16, 16)] = zeros_i

        def _p1(gi, cnt):
            r = gi // 8
            k = gi - r * 8
            v16 = idxf[r, pl.ds(k * 16, 16)]
            b16 = gi * 16 + iota
            for s in range(8):
                vloc = v16 - s * SUPW
                m = (vloc >= 0) & (vloc < SUPW)
                plsc.store_compressed(
                    supl.at[pl.ds(s * LCAP + cnt[s], 16)],
                    vloc * 4096 + b16, mask=m)
                pop = plsc.all_reduce_population_count(m)
                cnt = cnt + jnp.where(iota == s, pop, 0)
            return cnt

        cnt1 = lax.fori_loop(0, 256, _p1, zeros_i)

        ocopies = []
        for s in range(8):
            ns = cnt1[s]
            nsub = 8

            def _p2(gi, cnt, s=s, ns=ns, nsub=nsub):
                li = gi * 16 + iota
                lim = jnp.minimum(li, ns - 1)
                valid = li < ns
                p = plsc.load_gather(supl, [s * LCAP + lim])
                vl = p // 4096
                b = p - vl * 4096
                for t in range(nsub):
                    vs = vl - t * CW
                    m = valid & (vs >= 0) & (vs < CW)
                    plsc.store_compressed(
                        subl.at[pl.ds(t * LCAP + cnt[t], 16)],
                        vs * 4096 + b, mask=m)
                    pop = plsc.all_reduce_population_count(m)
                    cnt = cnt + jnp.where(iota == t, pop, 0)
                return cnt

            cnt2 = lax.fori_loop(0, (ns + 15) // 16, _p2, zeros_i)
            plsc.store_scatter(bcnt_v, [s * 8 + iota], cnt2,
                               mask=iota < nsub)
            ocopies.append(pltpu.async_copy(
                subl, lists_out.at[pl.ds((f * 64 + s * 8) * LCAP, 8 * LCAP)],
                osem))
            ocopies[-1].wait()

        pltpu.sync_copy(bcnt_v, cnt_out.at[pl.ds(f * 64, 64)])


@functools.partial(
    pl.kernel,
    mesh=_mesh,
    compiler_params=pltpu.CompilerParams(
        needs_layout_passes=False, use_tc_tiling_on_sc=True),
    out_type=jax.ShapeDtypeStruct((B * F, 128), jnp.float32),
    scratch_types=[
        pltpu.VMEM((4, 8, CW), jnp.float32),
        pltpu.VMEM((4, 8, CW), jnp.float32),
        pltpu.VMEM((4, 8, TAIL), jnp.float32),
        pltpu.VMEM((LCAP,), jnp.int32),
        pltpu.VMEM((64,), jnp.int32),
        pltpu.VMEM((16, 128), jnp.float32),
        pltpu.VMEM((16, 128), jnp.float32),
        pltpu.VMEM((16, 128), jnp.float32),
        pltpu.VMEM((16, 128), jnp.float32),
        pltpu.VMEM((16,), jnp.int32),
        pltpu.VMEM((16,), jnp.int32),
        pltpu.VMEM((16,), jnp.int32),
        pltpu.VMEM((16,), jnp.int32),
        pltpu.SemaphoreType.DMA,
        pltpu.SemaphoreType.DMA,
        pltpu.SemaphoreType.DMA,
        pltpu.SemaphoreType.DMA,
        pltpu.SemaphoreType.DMA,
        pltpu.SemaphoreType.DMA,
        pltpu.SemaphoreType.DMA,
        pltpu.SemaphoreType.DMA,
        pltpu.SemaphoreType.DMA,
    ],
)
def _gather_kernel(embn, emb_tail, lists_in, cnt_in, rows_out,
                   slab_a, slab_b, tslab, list_v, cnt_v,
                   stage0, stage1, stage2, stage3,
                   ridx0, ridx1, ridx2, ridx3,
                   sem_a, sem_b, tsem, csem, lsem,
                   ssem0, ssem1, ssem2, ssem3):
    w = lax.axis_index("s") * NC + lax.axis_index("c")
    c0 = w
    c1 = w + 32
    iota = lax.iota(jnp.int32, 16)

    NRING = 4
    stages = (stage0, stage1, stage2, stage3)
    ridxs = (ridx0, ridx1, ridx2, ridx3)
    ssems = (ssem0, ssem1, ssem2, ssem3)

    def slab_issue(c, f, buf, sem):
        @pl.when(c < NCH - 1)
        def _():
            pltpu.async_copy(
                embn.at[pl.ds(f * 4, 4), :,
                        pl.ds(pl.multiple_of(c * CW, 128), CW)], buf, sem)

        @pl.when(c == NCH - 1)
        def _():
            pltpu.async_copy(
                embn.at[pl.ds(f * 4, 4), :, pl.ds((NCH - 1) * CW, SW62)],
                buf.at[:, :, pl.ds(0, SW62)], sem)

    def slab_drain(c, buf, sem):
        @pl.when(c < NCH - 1)
        def _():
            pltpu.make_async_copy(
                embn.at[pl.ds(0, 4), :, pl.ds(0, CW)], buf, sem).wait()

        @pl.when(c == NCH - 1)
        def _():
            pltpu.make_async_copy(
                embn.at[pl.ds(0, 4), :, pl.ds(0, SW62)],
                buf.at[:, :, pl.ds(0, SW62)], sem).wait()

    def getn(c):
        grp = c // 16
        vec = cnt_v[pl.ds(pl.multiple_of(grp * 16, 16), 16)]
        return jnp.sum(jnp.where(iota == c - grp * 16, vec, 0))

    def serve(c, n, f, slab, flags):
        sw = jnp.where(c == NCH - 1, SW62, CW)
        base = (f * 64 + c) * LCAP
        pltpu.async_copy(
            lists_in.at[pl.ds(base, 512)], list_v.at[pl.ds(0, 512)],
            lsem).wait()

        @pl.when(n > 512)
        def _():
            pltpu.async_copy(
                lists_in.at[pl.ds(base + 512, 1536)],
                list_v.at[pl.ds(512, 1536)], lsem).wait()

        @pl.when(n > 2048)
        def _():
            pltpu.async_copy(
                lists_in.at[pl.ds(base + 2048, 2048)],
                list_v.at[pl.ds(2048, 2048)], lsem).wait()

        ng = (n + 15) // 16

        def _serve_wave(t, flags):
            newflags = []
            for s in range(NRING):
                gi = t * NRING + s
                stage, ridx, ssem = stages[s], ridxs[s], ssems[s]
                fired = flags[s]

                @pl.when(gi < ng)
                def _():
                    @pl.when(fired > 0)
                    def _():
                        pltpu.make_async_copy(
                            stage, rows_out.at[ridx], ssem).wait()

                    li = jnp.minimum(gi * 16 + iota, n - 1)
                    p16 = plsc.load_gather(list_v, [li])
                    vloc = p16 // 4096
                    b16 = p16 - vloc * 4096
                    in_main = vloc < sw
                    vmain = jnp.minimum(vloc, sw - 1)
                    vtail = jnp.clip(vloc - sw + TOFF, 0, TAIL - 1)
                    for dt in range(4):
                        dtv = jnp.full((16,), dt, jnp.int32)
                        for ds in range(8):
                            dsv = jnp.full((16,), ds, jnp.int32)
                            e16 = plsc.load_gather(slab, [dtv, dsv, vmain])
                            t16 = plsc.load_gather(tslab, [dtv, dsv, vtail])
                            e16 = jnp.where(in_main, e16, t16)
                            plsc.store_scatter(
                                stage,
                                [iota,
                                 jnp.full((16,), dt * 8 + ds, jnp.int32)],
                                e16)
                    ridx[...] = b16 * F + f
                    pltpu.async_copy(stage, rows_out.at[ridx], ssem)

                newflags.append(jnp.where(gi < ng, jnp.int32(1), fired))
            return tuple(newflags)

        return lax.fori_loop(0, (ng + NRING - 1) // NRING, _serve_wave,
                             flags)

    slab_issue(c0, 0, slab_a, sem_a)

    def _field(f, flags):
        pltpu.async_copy(cnt_in.at[pl.ds(f * 64, 64)], cnt_v, csem).wait()
        n0 = getn(c0)
        n1 = getn(c1)
        slab_issue(c1, f, slab_b, sem_b)

        @pl.when(c1 == NCH - 1)
        def _():
            pltpu.async_copy(emb_tail.at[pl.ds(f * 4, 4)], tslab,
                             tsem).wait()

        slab_drain(c0, slab_a, sem_a)
        flags = serve(c0, n0, f, slab_a, flags)

        @pl.when(f < F - 1)
        def _():
            slab_issue(c0, f + 1, slab_a, sem_a)

        slab_drain(c1, slab_b, sem_b)
        flags = serve(c1, n1, f, slab_b, flags)
        return flags

    zero = jnp.int32(0)
    flags = lax.fori_loop(0, F, _field, (zero, zero, zero, zero))

    for s in range(NRING):
        @pl.when(flags[s] > 0)
        def _():
            pltpu.make_async_copy(
                stages[s], rows_out.at[ridxs[s]], ssems[s]).wait()


@functools.partial(
    pl.kernel,
    mesh=_mesh,
    compiler_params=pltpu.CompilerParams(
        needs_layout_passes=False, use_tc_tiling_on_sc=True),
    out_type=jax.ShapeDtypeStruct((B,), jnp.float32),
    scratch_types=[
        pltpu.VMEM((16 * F, 128), jnp.float32),
        pltpu.VMEM((16 * F, 128), jnp.float32),
        pltpu.VMEM((BPW,), jnp.float32),
        pltpu.SemaphoreType.DMA,
        pltpu.SemaphoreType.DMA,
    ],
)
def _stats_kernel(rows_in, acc_out, buf0, buf1, acc_v, sem0, sem1):
    w = lax.axis_index("s") * NC + lax.axis_index("c")
    base_row = w * RPW

    iota = lax.iota(jnp.int32, 16)
    zeros_f = jnp.zeros((16,), jnp.float32)
    bufs = (buf0, buf1)
    sems = (sem0, sem1)

    copies = []
    for g in range(2):
        copies.append(pltpu.async_copy(
            rows_in.at[pl.ds(base_row + g * (16 * F), 16 * F)],
            bufs[g], sems[g]))

    for g in range(GROUPS):
        s = g % 2
        buf = bufs[s]
        copies[g].wait()

        row_idx = [iota * F + f for f in range(F)]

        def _lane(d, acc):
            dvec = jnp.full((16,), d, jnp.int32)
            sa = zeros_f
            qa = zeros_f
            for f in range(F):
                e = plsc.load_gather(buf, [row_idx[f], dvec])
                sa = sa + e
                qa = qa + e * e
            return acc + (sa * sa - qa)

        acc = lax.fori_loop(0, D, _lane, zeros_f)
        acc_v[pl.ds(g * 16, 16)] = 0.5 * acc

        if g + 2 < GROUPS:
            copies.append(pltpu.async_copy(
                rows_in.at[pl.ds(base_row + (g + 2) * (16 * F), 16 * F)],
                bufs[s], sems[s]))

    pltpu.sync_copy(acc_v, acc_out.at[pl.ds(w * BPW, BPW)])


@functools.partial(
    pl.kernel,
    mesh=_mesh,
    compiler_params=pltpu.CompilerParams(
        needs_layout_passes=False, use_tc_tiling_on_sc=False),
    out_type=jax.ShapeDtypeStruct((B,), jnp.float32),
    scratch_types=[
        pltpu.VMEM((NCHUNK, CHUNK), jnp.int32),
        pltpu.VMEM((RPW,), jnp.float32),
        pltpu.VMEM((BPW,), jnp.float32),
        pltpu.VMEM((BPW,), jnp.float32),
        pltpu.SemaphoreType.DMA,
        pltpu.SemaphoreType.DMA,
    ],
)
def _bias_kernel(idx_hbm, bias_hbm, acc_hbm, out1_hbm,
                 idx_v, bias_v, acc_v, out_v, bsem, asem):
    wid = lax.axis_index("s") * NC + lax.axis_index("c")
    base_row = wid * RPW
    base_samp = wid * BPW

    pltpu.sync_copy(idx_hbm.at[wid], idx_v)
    acc_cp = pltpu.async_copy(acc_hbm.at[pl.ds(base_samp, BPW)], acc_v, asem)

    iota = lax.iota(jnp.int32, 16)

    def _add_off(t, carry):
        j = t // 8
        col = (t - j * 8) * 16
        rvec = (base_row + t * 16) + iota
        fvec = lax.rem(rvec, F)
        idx_v[j, pl.ds(col, 16)] = idx_v[j, pl.ds(col, 16)] + fvec * V
        return carry

    lax.fori_loop(0, NCHUNK * 8, _add_off, 0)

    bcopies = []
    for j in range(NCHUNK):
        bcopies.append(pltpu.async_copy(
            bias_hbm.at[idx_v.at[j]], bias_v.at[pl.ds(j * CHUNK, CHUNK)],
            bsem))
    for cp in bcopies:
        cp.wait()
    acc_cp.wait()

    zeros_f = jnp.zeros((16,), jnp.float32)
    stride = iota * F

    def _group(g, carry):
        rb = g * (16 * F)
        bias_acc = zeros_f
        for f in range(F):
            bias_acc = bias_acc + plsc.load_gather(bias_v, [stride + rb + f])
        out_v[pl.ds(g * 16, 16)] = bias_acc + acc_v[pl.ds(g * 16, 16)]
        return carry

    lax.fori_loop(0, GROUPS, _group, 0)

    pltpu.sync_copy(out_v, out1_hbm.at[pl.ds(base_samp, BPW)])


def kernel(field_indices, emb_tables, bias_tables):
    idx3 = field_indices.T.reshape(F, 32, 128)
    embn = emb_tables.transpose(0, 2, 1).reshape(F * D // 8, 8, V)
    emb_tail = (emb_tables[:, V - TAIL:, :]
                .transpose(0, 2, 1).reshape(F * D // 8, 8, TAIL))
    lists, cnts = _bucket_kernel(idx3)
    rows = _gather_kernel(embn, emb_tail, lists, cnts)
    acc = _stats_kernel(rows)
    idx_w = field_indices.reshape(NW, NCHUNK, CHUNK)
    bias_flat = bias_tables.reshape(F * V)
    out1 = _bias_kernel(idx_w, bias_flat, acc)
    embeds = rows[:, :D].reshape(B, F, D)
    return (out1.reshape(B, 1), embeds)


# speedup vs baseline: 1.1013x; 1.0028x over previous
import functools

import jax
import jax.numpy as jnp
from jax import lax
from jax.experimental import pallas as pl
from jax.experimental.pallas import tpu as pltpu
from jax.experimental.pallas import tpu_sc as plsc

F = 26
V = 100000
D = 32
B = 4096

NC = 2
NS = 16
NW = NC * NS

CW = 1664
NCH = 61
SW62 = 128
TAIL = 128
TOFF = (NCH - 1) * CW + SW62 - (V - TAIL)
SUPW = 8 * CW
LCAP = 4112

BPW = B // NW
RPW = BPW * F
CHUNK = 128
NCHUNK = RPW // CHUNK
GROUPS = BPW // 16

_mesh = plsc.VectorSubcoreMesh(core_axis_name="c", subcore_axis_name="s")


@functools.partial(
    pl.kernel,
    mesh=_mesh,
    compiler_params=pltpu.CompilerParams(
        needs_layout_passes=False, use_tc_tiling_on_sc=True),
    out_type=[
        jax.ShapeDtypeStruct((F * 64 * LCAP,), jnp.int32),
        jax.ShapeDtypeStruct((F * 64,), jnp.int32),
    ],
    scratch_types=[
        pltpu.VMEM((32, 128), jnp.int32),
        pltpu.VMEM((8 * LCAP,), jnp.int32),
        pltpu.VMEM((8 * LCAP,), jnp.int32),
        pltpu.VMEM((64,), jnp.int32),
        pltpu.SemaphoreType.DMA,
        pltpu.SemaphoreType.DMA,
    ],
)
def _bucket_kernel(idx3, lists_out, cnt_out, idxf, supl, subl, bcnt_v,
                   isem, osem):
    w = lax.axis_index("s") * NC + lax.axis_index("c")
    iota = lax.iota(jnp.int32, 16)
    zeros_i = jnp.zeros((16,), jnp.int32)

    @pl.when(w < F)
    def _():
        f = w
        pltpu.async_copy(idx3.at[f], idxf, isem).wait()
        for q in range(4):
            bcnt_v[pl.ds(q * 16, 16)] = zeros_i

        def _p1(gi, cnt):
            r = gi // 8
            k = gi - r * 8
            v16 = idxf[r, pl.ds(k * 16, 16)]
            b16 = gi * 16 + iota
            for s in range(8):
                vloc = v16 - s * SUPW
                m = (vloc >= 0) & (vloc < SUPW)
                plsc.store_compressed(
                    supl.at[pl.ds(s * LCAP + cnt[s], 16)],
                    vloc * 4096 + b16, mask=m)
                pop = plsc.all_reduce_population_count(m)
                cnt = cnt + jnp.where(iota == s, pop, 0)
            return cnt

        cnt1 = lax.fori_loop(0, 256, _p1, zeros_i)

        ocopies = []
        for s in range(8):
            ns = cnt1[s]
            nsub = 8

            def _p2(gi, cnt, s=s, ns=ns, nsub=nsub):
                li = gi * 16 + iota
                lim = jnp.minimum(li, ns - 1)
                valid = li < ns
                p = plsc.load_gather(supl, [s * LCAP + lim])
                vl = p // 4096
                b = p - vl * 4096
                for t in range(nsub):
                    vs = vl - t * CW
                    m = valid & (vs >= 0) & (vs < CW)
                    plsc.store_compressed(
                        subl.at[pl.ds(t * LCAP + cnt[t], 16)],
                        vs * 4096 + b, mask=m)
                    pop = plsc.all_reduce_population_count(m)
                    cnt = cnt + jnp.where(iota == t, pop, 0)
                return cnt

            cnt2 = lax.fori_loop(0, (ns + 15) // 16, _p2, zeros_i)
            plsc.store_scatter(bcnt_v, [s * 8 + iota], cnt2,
                               mask=iota < nsub)
            ocopies.append(pltpu.async_copy(
                subl, lists_out.at[pl.ds((f * 64 + s * 8) * LCAP, 8 * LCAP)],
                osem))
            ocopies[-1].wait()

        pltpu.sync_copy(bcnt_v, cnt_out.at[pl.ds(f * 64, 64)])


@functools.partial(
    pl.kernel,
    mesh=_mesh,
    compiler_params=pltpu.CompilerParams(
        needs_layout_passes=False, use_tc_tiling_on_sc=True),
    out_type=jax.ShapeDtypeStruct((B * F, 128), jnp.float32),
    scratch_types=[
        pltpu.VMEM((4, 8, CW), jnp.float32),
        pltpu.VMEM((4, 8, CW), jnp.float32),
        pltpu.VMEM((4, 8, TAIL), jnp.float32),
        pltpu.VMEM((LCAP,), jnp.int32),
        pltpu.VMEM((64,), jnp.int32),
        pltpu.VMEM((16, 128), jnp.float32),
        pltpu.VMEM((16, 128), jnp.float32),
        pltpu.VMEM((16, 128), jnp.float32),
        pltpu.VMEM((16, 128), jnp.float32),
        pltpu.VMEM((16,), jnp.int32),
        pltpu.VMEM((16,), jnp.int32),
        pltpu.VMEM((16,), jnp.int32),
        pltpu.VMEM((16,), jnp.int32),
        pltpu.SemaphoreType.DMA,
        pltpu.SemaphoreType.DMA,
        pltpu.SemaphoreType.DMA,
        pltpu.SemaphoreType.DMA,
        pltpu.SemaphoreType.DMA,
        pltpu.SemaphoreType.DMA,
        pltpu.SemaphoreType.DMA,
        pltpu.SemaphoreType.DMA,
        pltpu.SemaphoreType.DMA,
    ],
)
def _gather_kernel(embn, emb_tail, lists_in, cnt_in, rows_out,
                   slab_a, slab_b, tslab, list_v, cnt_v,
                   stage0, stage1, stage2, stage3,
                   ridx0, ridx1, ridx2, ridx3,
                   sem_a, sem_b, tsem, csem, lsem,
                   ssem0, ssem1, ssem2, ssem3):
    w = lax.axis_index("s") * NC + lax.axis_index("c")
    c0 = w
    c1 = w + 32
    iota = lax.iota(jnp.int32, 16)

    NRING = 4
    stages = (stage0, stage1, stage2, stage3)
    ridxs = (ridx0, ridx1, ridx2, ridx3)
    ssems = (ssem0, ssem1, ssem2, ssem3)

    def slab_issue(c, f, buf, sem):
        @pl.when(c < NCH - 1)
        def _():
            pltpu.async_copy(
                embn.at[pl.ds(f * 4, 4), :,
                        pl.ds(pl.multiple_of(c * CW, 128), CW)], buf, sem)

        @pl.when(c == NCH - 1)
        def _():
            pltpu.async_copy(
                embn.at[pl.ds(f * 4, 4), :, pl.ds((NCH - 1) * CW, SW62)],
                buf.at[:, :, pl.ds(0, SW62)], sem)

    def slab_drain(c, buf, sem):
        @pl.when(c < NCH - 1)
        def _():
            pltpu.make_async_copy(
                embn.at[pl.ds(0, 4), :, pl.ds(0, CW)], buf, sem).wait()

        @pl.when(c == NCH - 1)
        def _():
            pltpu.make_async_copy(
                embn.at[pl.ds(0, 4), :, pl.ds(0, SW62)],
                buf.at[:, :, pl.ds(0, SW62)], sem).wait()

    def getn(c):
        grp = c // 16
        vec = cnt_v[pl.ds(pl.multiple_of(grp * 16, 16), 16)]
        return jnp.sum(jnp.where(iota == c - grp * 16, vec, 0))

    def serve(c, n, f, slab, flags):
        sw = jnp.where(c == NCH - 1, SW62, CW)
        base = (f * 64 + c) * LCAP
        pltpu.async_copy(
            lists_in.at[pl.ds(base, 512)], list_v.at[pl.ds(0, 512)],
            lsem).wait()

        @pl.when(n > 512)
        def _():
            pltpu.async_copy(
                lists_in.at[pl.ds(base + 512, 1536)],
                list_v.at[pl.ds(512, 1536)], lsem).wait()

        @pl.when(n > 2048)
        def _():
            pltpu.async_copy(
                lists_in.at[pl.ds(base + 2048, 2048)],
                list_v.at[pl.ds(2048, 2048)], lsem).wait()

        ng = (n + 15) // 16

        def _serve_wave(t, flags):
            newflags = []
            for s in range(NRING):
                gi = t * NRING + s
                stage, ridx, ssem = stages[s], ridxs[s], ssems[s]
                fired = flags[s]

                @pl.when(gi < ng)
                def _():
                    @pl.when(fired > 0)
                    def _():
                        pltpu.make_async_copy(
                            stage, rows_out.at[ridx], ssem).wait()

                    li = jnp.minimum(gi * 16 + iota, n - 1)
                    p16 = plsc.load_gather(list_v, [li])
                    vloc = p16 // 4096
                    b16 = p16 - vloc * 4096
                    in_main = vloc < sw
                    vmain = jnp.minimum(vloc, sw - 1)
                    vtail = jnp.clip(vloc - sw + TOFF, 0, TAIL - 1)
                    for dt in range(4):
                        dtv = jnp.full((16,), dt, jnp.int32)
                        for ds in range(8):
                            dsv = jnp.full((16,), ds, jnp.int32)
                            e16 = plsc.load_gather(slab, [dtv, dsv, vmain])
                            t16 = plsc.load_gather(tslab, [dtv, dsv, vtail])
                            e16 = jnp.where(in_main, e16, t16)
                            plsc.store_scatter(
                                stage,
                                [iota,
                                 jnp.full((16,), dt * 8 + ds, jnp.int32)],
                                e16)
                    ridx[...] = b16 * F + f
                    pltpu.async_copy(stage, rows_out.at[ridx], ssem)

                newflags.append(jnp.where(gi < ng, jnp.int32(1), fired))
            return tuple(newflags)

        return lax.fori_loop(0, (ng + NRING - 1) // NRING, _serve_wave,
                             flags)

    slab_issue(c0, 0, slab_a, sem_a)

    def _field(f, flags):
        pltpu.async_copy(cnt_in.at[pl.ds(f * 64, 64)], cnt_v, csem).wait()
        n0 = getn(c0)
        n1 = getn(c1)
        slab_issue(c1, f, slab_b, sem_b)

        @pl.when(c1 == NCH - 1)
        def _():
            pltpu.async_copy(emb_tail.at[pl.ds(f * 4, 4)], tslab,
                             tsem).wait()

        slab_drain(c0, slab_a, sem_a)
        flags = serve(c0, n0, f, slab_a, flags)

        @pl.when(f < F - 1)
        def _():
            slab_issue(c0, f + 1, slab_a, sem_a)

        slab_drain(c1, slab_b, sem_b)
        flags = serve(c1, n1, f, slab_b, flags)
        return flags

    zero = jnp.int32(0)
    flags = lax.fori_loop(0, F, _field, (zero, zero, zero, zero))

    for s in range(NRING):
        @pl.when(flags[s] > 0)
        def _():
            pltpu.make_async_copy(
                stages[s], rows_out.at[ridxs[s]], ssems[s]).wait()


@functools.partial(
    pl.kernel,
    mesh=_mesh,
    compiler_params=pltpu.CompilerParams(
        needs_layout_passes=False, use_tc_tiling_on_sc=True),
    out_type=jax.ShapeDtypeStruct((B,), jnp.float32),
    scratch_types=[
        pltpu.VMEM((16 * F, 128), jnp.float32),
        pltpu.VMEM((16 * F, 128), jnp.float32),
        pltpu.VMEM((BPW,), jnp.float32),
        pltpu.SemaphoreType.DMA,
        pltpu.SemaphoreType.DMA,
    ],
)
def _stats_kernel(rows_in, acc_out, buf0, buf1, acc_v, sem0, sem1):
    w = lax.axis_index("s") * NC + lax.axis_index("c")
    base_row = w * RPW

    iota = lax.iota(jnp.int32, 16)
    zeros_f = jnp.zeros((16,), jnp.float32)
    bufs = (buf0, buf1)
    sems = (sem0, sem1)

    copies = []
    for g in range(2):
        copies.append(pltpu.async_copy(
            rows_in.at[pl.ds(base_row + g * (16 * F), 16 * F)],
            bufs[g], sems[g]))

    for g in range(GROUPS):
        s = g % 2
        buf = bufs[s]
        copies[g].wait()

        row_idx = [iota * F + f for f in range(F)]

        def _lane(d, acc):
            dvec = jnp.full((16,), d, jnp.int32)
            sa0 = sa1 = qa0 = qa1 = zeros_f
            for f in range(0, F, 2):
                e0 = plsc.load_gather(buf, [row_idx[f], dvec])
                e1 = plsc.load_gather(buf, [row_idx[f + 1], dvec])
                sa0 = sa0 + e0
                qa0 = qa0 + e0 * e0
                sa1 = sa1 + e1
                qa1 = qa1 + e1 * e1
            sa = sa0 + sa1
            return acc + (sa * sa - (qa0 + qa1))

        acc = lax.fori_loop(0, D, _lane, zeros_f)
        acc_v[pl.ds(g * 16, 16)] = 0.5 * acc

        if g + 2 < GROUPS:
            copies.append(pltpu.async_copy(
                rows_in.at[pl.ds(base_row + (g + 2) * (16 * F), 16 * F)],
                bufs[s], sems[s]))

    pltpu.sync_copy(acc_v, acc_out.at[pl.ds(w * BPW, BPW)])


@functools.partial(
    pl.kernel,
    mesh=_mesh,
    compiler_params=pltpu.CompilerParams(
        needs_layout_passes=False, use_tc_tiling_on_sc=False),
    out_type=jax.ShapeDtypeStruct((B,), jnp.float32),
    scratch_types=[
        pltpu.VMEM((NCHUNK, CHUNK), jnp.int32),
        pltpu.VMEM((RPW,), jnp.float32),
        pltpu.VMEM((BPW,), jnp.float32),
        pltpu.VMEM((BPW,), jnp.float32),
        pltpu.SemaphoreType.DMA,
        pltpu.SemaphoreType.DMA,
    ],
)
def _bias_kernel(idx_hbm, bias_hbm, acc_hbm, out1_hbm,
                 idx_v, bias_v, acc_v, out_v, bsem, asem):
    wid = lax.axis_index("s") * NC + lax.axis_index("c")
    base_row = wid * RPW
    base_samp = wid * BPW

    pltpu.sync_copy(idx_hbm.at[wid], idx_v)
    acc_cp = pltpu.async_copy(acc_hbm.at[pl.ds(base_samp, BPW)], acc_v, asem)

    iota = lax.iota(jnp.int32, 16)

    def _add_off(t, carry):
        j = t // 8
        col = (t - j * 8) * 16
        rvec = (base_row + t * 16) + iota
        fvec = lax.rem(rvec, F)
        idx_v[j, pl.ds(col, 16)] = idx_v[j, pl.ds(col, 16)] + fvec * V
        return carry

    lax.fori_loop(0, NCHUNK * 8, _add_off, 0)

    bcopies = []
    for j in range(NCHUNK):
        bcopies.append(pltpu.async_copy(
            bias_hbm.at[idx_v.at[j]], bias_v.at[pl.ds(j * CHUNK, CHUNK)],
            bsem))
    for cp in bcopies:
        cp.wait()
    acc_cp.wait()

    zeros_f = jnp.zeros((16,), jnp.float32)
    stride = iota * F

    def _group(g, carry):
        rb = g * (16 * F)
        bias_acc = zeros_f
        for f in range(F):
            bias_acc = bias_acc + plsc.load_gather(bias_v, [stride + rb + f])
        out_v[pl.ds(g * 16, 16)] = bias_acc + acc_v[pl.ds(g * 16, 16)]
        return carry

    lax.fori_loop(0, GROUPS, _group, 0)

    pltpu.sync_copy(out_v, out1_hbm.at[pl.ds(base_samp, BPW)])


def kernel(field_indices, emb_tables, bias_tables):
    idx3 = field_indices.T.reshape(F, 32, 128)
    embn = emb_tables.transpose(0, 2, 1).reshape(F * D // 8, 8, V)
    emb_tail = (emb_tables[:, V - TAIL:, :]
                .transpose(0, 2, 1).reshape(F * D // 8, 8, TAIL))
    lists, cnts = _bucket_kernel(idx3)
    rows = _gather_kernel(embn, emb_tail, lists, cnts)
    acc = _stats_kernel(rows)
    idx_w = field_indices.reshape(NW, NCHUNK, CHUNK)
    bias_flat = bias_tables.reshape(F * V)
    out1 = _bias_kernel(idx_w, bias_flat, acc)
    embeds = rows[:, :D].reshape(B, F, D)
    return (out1.reshape(B, 1), embeds)


# speedup vs baseline: 1.1483x; 1.0426x over previous
import functools

import jax
import jax.numpy as jnp
from jax import lax
from jax.experimental import pallas as pl
from jax.experimental.pallas import tpu as pltpu
from jax.experimental.pallas import tpu_sc as plsc

F = 26
V = 100000
D = 32
B = 4096

NC = 2
NS = 16
NW = NC * NS

CW = 1664
NCH = 61
SW62 = 128
TAIL = 128
TOFF = (NCH - 1) * CW + SW62 - (V - TAIL)
SUPW = 8 * CW
LCAP = 4112

BPW = B // NW
RPW = BPW * F
CHUNK = 128
NCHUNK = RPW // CHUNK
GROUPS = BPW // 16

_mesh = plsc.VectorSubcoreMesh(core_axis_name="c", subcore_axis_name="s")


@functools.partial(
    pl.kernel,
    mesh=_mesh,
    compiler_params=pltpu.CompilerParams(
        needs_layout_passes=False, use_tc_tiling_on_sc=True),
    out_type=[
        jax.ShapeDtypeStruct((F * 64 * LCAP,), jnp.int32),
        jax.ShapeDtypeStruct((F * 64,), jnp.int32),
    ],
    scratch_types=[
        pltpu.VMEM((32, 128), jnp.int32),
        pltpu.VMEM((8 * LCAP,), jnp.int32),
        pltpu.VMEM((8 * LCAP,), jnp.int32),
        pltpu.VMEM((64,), jnp.int32),
        pltpu.SemaphoreType.DMA,
        pltpu.SemaphoreType.DMA,
    ],
)
def _bucket_kernel(idx3, lists_out, cnt_out, idxf, supl, subl, bcnt_v,
                   isem, osem):
    w = lax.axis_index("s") * NC + lax.axis_index("c")
    iota = lax.iota(jnp.int32, 16)
    zeros_i = jnp.zeros((16,), jnp.int32)

    @pl.when(w < F)
    def _():
        f = w
        pltpu.async_copy(idx3.at[f], idxf, isem).wait()
        for q in range(4):
            bcnt_v[pl.ds(q * 16, 16)] = zeros_i

        def _p1(gi, cnt):
            r = gi // 8
            k = gi - r * 8
            v16 = idxf[r, pl.ds(k * 16, 16)]
            b16 = gi * 16 + iota
            for s in range(8):
                vloc = v16 - s * SUPW
                m = (vloc >= 0) & (vloc < SUPW)
                plsc.store_compressed(
                    supl.at[pl.ds(s * LCAP + cnt[s], 16)],
                    vloc * 4096 + b16, mask=m)
                pop = plsc.all_reduce_population_count(m)
                cnt = cnt + jnp.where(iota == s, pop, 0)
            return cnt

        cnt1 = lax.fori_loop(0, 256, _p1, zeros_i)

        ocopies = []
        for s in range(8):
            ns = cnt1[s]
            nsub = 8

            def _p2(gi, cnt, s=s, ns=ns, nsub=nsub):
                li = gi * 16 + iota
                lim = jnp.minimum(li, ns - 1)
                valid = li < ns
                p = plsc.load_gather(supl, [s * LCAP + lim])
                vl = p // 4096
                b = p - vl * 4096
                for t in range(nsub):
                    vs = vl - t * CW
                    m = valid & (vs >= 0) & (vs < CW)
                    plsc.store_compressed(
                        subl.at[pl.ds(t * LCAP + cnt[t], 16)],
                        vs * 4096 + b, mask=m)
                    pop = plsc.all_reduce_population_count(m)
                    cnt = cnt + jnp.where(iota == t, pop, 0)
                return cnt

            cnt2 = lax.fori_loop(0, (ns + 15) // 16, _p2, zeros_i)
            plsc.store_scatter(bcnt_v, [s * 8 + iota], cnt2,
                               mask=iota < nsub)
            ocopies.append(pltpu.async_copy(
                subl, lists_out.at[pl.ds((f * 64 + s * 8) * LCAP, 8 * LCAP)],
                osem))
            ocopies[-1].wait()

        pltpu.sync_copy(bcnt_v, cnt_out.at[pl.ds(f * 64, 64)])


@functools.partial(
    pl.kernel,
    mesh=_mesh,
    compiler_params=pltpu.CompilerParams(
        needs_layout_passes=False, use_tc_tiling_on_sc=True),
    out_type=jax.ShapeDtypeStruct((B * F, 128), jnp.float32),
    scratch_types=[
        pltpu.VMEM((4, 8, CW), jnp.float32),
        pltpu.VMEM((4, 8, CW), jnp.float32),
        pltpu.VMEM((4, 8, TAIL), jnp.float32),
        pltpu.VMEM((LCAP,), jnp.int32),
        pltpu.VMEM((LCAP,), jnp.int32),
        pltpu.VMEM((64,), jnp.int32),
        pltpu.VMEM((16, 128), jnp.float32),
        pltpu.VMEM((16, 128), jnp.float32),
        pltpu.VMEM((16, 128), jnp.float32),
        pltpu.VMEM((16, 128), jnp.float32),
        pltpu.VMEM((16,), jnp.int32),
        pltpu.VMEM((16,), jnp.int32),
        pltpu.VMEM((16,), jnp.int32),
        pltpu.VMEM((16,), jnp.int32),
        pltpu.SemaphoreType.DMA,
        pltpu.SemaphoreType.DMA,
        pltpu.SemaphoreType.DMA,
        pltpu.SemaphoreType.DMA,
        pltpu.SemaphoreType.DMA,
        pltpu.SemaphoreType.DMA,
        pltpu.SemaphoreType.DMA,
        pltpu.SemaphoreType.DMA,
        pltpu.SemaphoreType.DMA,
        pltpu.SemaphoreType.DMA,
    ],
)
def _gather_kernel(embn, emb_tail, lists_in, cnt_in, rows_out,
                   slab_a, slab_b, tslab, list_v0, list_v1, cnt_v,
                   stage0, stage1, stage2, stage3,
                   ridx0, ridx1, ridx2, ridx3,
                   sem_a, sem_b, tsem, csem, lsem0, lsem1,
                   ssem0, ssem1, ssem2, ssem3):
    w = lax.axis_index("s") * NC + lax.axis_index("c")
    c0 = w
    c1 = w + 32
    iota = lax.iota(jnp.int32, 16)

    NRING = 4
    stages = (stage0, stage1, stage2, stage3)
    ridxs = (ridx0, ridx1, ridx2, ridx3)
    ssems = (ssem0, ssem1, ssem2, ssem3)

    def slab_issue(c, f, buf, sem):
        @pl.when(c < NCH - 1)
        def _():
            pltpu.async_copy(
                embn.at[pl.ds(f * 4, 4), :,
                        pl.ds(pl.multiple_of(c * CW, 128), CW)], buf, sem)

        @pl.when(c == NCH - 1)
        def _():
            pltpu.async_copy(
                embn.at[pl.ds(f * 4, 4), :, pl.ds((NCH - 1) * CW, SW62)],
                buf.at[:, :, pl.ds(0, SW62)], sem)

    def slab_drain(c, buf, sem):
        @pl.when(c < NCH - 1)
        def _():
            pltpu.make_async_copy(
                embn.at[pl.ds(0, 4), :, pl.ds(0, CW)], buf, sem).wait()

        @pl.when(c == NCH - 1)
        def _():
            pltpu.make_async_copy(
                embn.at[pl.ds(0, 4), :, pl.ds(0, SW62)],
                buf.at[:, :, pl.ds(0, SW62)], sem).wait()

    def getn(c):
        grp = c // 16
        vec = cnt_v[pl.ds(pl.multiple_of(grp * 16, 16), 16)]
        return jnp.sum(jnp.where(iota == c - grp * 16, vec, 0))

    def list_issue(c, n, f, list_v, lsem):
        base = (f * 64 + c) * LCAP
        pltpu.async_copy(
            lists_in.at[pl.ds(base, 512)], list_v.at[pl.ds(0, 512)], lsem)

        @pl.when(n > 512)
        def _():
            pltpu.async_copy(
                lists_in.at[pl.ds(base + 512, 1536)],
                list_v.at[pl.ds(512, 1536)], lsem)

        @pl.when(n > 2048)
        def _():
            pltpu.async_copy(
                lists_in.at[pl.ds(base + 2048, 2048)],
                list_v.at[pl.ds(2048, 2048)], lsem)

    def list_drain(n, list_v, lsem):
        pltpu.make_async_copy(
            lists_in.at[pl.ds(0, 512)], list_v.at[pl.ds(0, 512)],
            lsem).wait()

        @pl.when(n > 512)
        def _():
            pltpu.make_async_copy(
                lists_in.at[pl.ds(512, 1536)],
                list_v.at[pl.ds(512, 1536)], lsem).wait()

        @pl.when(n > 2048)
        def _():
            pltpu.make_async_copy(
                lists_in.at[pl.ds(2048, 2048)],
                list_v.at[pl.ds(2048, 2048)], lsem).wait()

    def serve(c, n, f, slab, flags, list_v):
        sw = jnp.where(c == NCH - 1, SW62, CW)
        ng = (n + 15) // 16

        def _serve_wave(t, flags):
            newflags = []
            for s in range(NRING):
                gi = t * NRING + s
                stage, ridx, ssem = stages[s], ridxs[s], ssems[s]
                fired = flags[s]

                @pl.when(gi < ng)
                def _():
                    @pl.when(fired > 0)
                    def _():
                        pltpu.make_async_copy(
                            stage, rows_out.at[ridx], ssem).wait()

                    li = jnp.minimum(gi * 16 + iota, n - 1)
                    p16 = plsc.load_gather(list_v, [li])
                    vloc = p16 // 4096
                    b16 = p16 - vloc * 4096
                    in_main = vloc < sw
                    vmain = jnp.minimum(vloc, sw - 1)
                    vtail = jnp.clip(vloc - sw + TOFF, 0, TAIL - 1)
                    for dt in range(4):
                        dtv = jnp.full((16,), dt, jnp.int32)
                        for ds in range(8):
                            dsv = jnp.full((16,), ds, jnp.int32)
                            e16 = plsc.load_gather(slab, [dtv, dsv, vmain])
                            t16 = plsc.load_gather(tslab, [dtv, dsv, vtail])
                            e16 = jnp.where(in_main, e16, t16)
                            plsc.store_scatter(
                                stage,
                                [iota,
                                 jnp.full((16,), dt * 8 + ds, jnp.int32)],
                                e16)
                    ridx[...] = b16 * F + f
                    pltpu.async_copy(stage, rows_out.at[ridx], ssem)

                newflags.append(jnp.where(gi < ng, jnp.int32(1), fired))
            return tuple(newflags)

        return lax.fori_loop(0, (ng + NRING - 1) // NRING, _serve_wave,
                             flags)

    slab_issue(c0, 0, slab_a, sem_a)

    def _field(f, flags):
        pltpu.async_copy(cnt_in.at[pl.ds(f * 64, 64)], cnt_v, csem).wait()
        n0 = getn(c0)
        n1 = getn(c1)
        slab_issue(c1, f, slab_b, sem_b)
        list_issue(c0, n0, f, list_v0, lsem0)
        list_issue(c1, n1, f, list_v1, lsem1)

        @pl.when(c1 == NCH - 1)
        def _():
            pltpu.async_copy(emb_tail.at[pl.ds(f * 4, 4)], tslab,
                             tsem).wait()

        slab_drain(c0, slab_a, sem_a)
        list_drain(n0, list_v0, lsem0)
        flags = serve(c0, n0, f, slab_a, flags, list_v0)

        @pl.when(f < F - 1)
        def _():
            slab_issue(c0, f + 1, slab_a, sem_a)

        slab_drain(c1, slab_b, sem_b)
        list_drain(n1, list_v1, lsem1)
        flags = serve(c1, n1, f, slab_b, flags, list_v1)
        return flags

    zero = jnp.int32(0)
    flags = lax.fori_loop(0, F, _field, (zero, zero, zero, zero))

    for s in range(NRING):
        @pl.when(flags[s] > 0)
        def _():
            pltpu.make_async_copy(
                stages[s], rows_out.at[ridxs[s]], ssems[s]).wait()


@functools.partial(
    pl.kernel,
    mesh=_mesh,
    compiler_params=pltpu.CompilerParams(
        needs_layout_passes=False, use_tc_tiling_on_sc=True),
    out_type=jax.ShapeDtypeStruct((B,), jnp.float32),
    scratch_types=[
        pltpu.VMEM((16 * F, 128), jnp.float32),
        pltpu.VMEM((16 * F, 128), jnp.float32),
        pltpu.VMEM((BPW,), jnp.float32),
        pltpu.SemaphoreType.DMA,
        pltpu.SemaphoreType.DMA,
    ],
)
def _stats_kernel(rows_in, acc_out, buf0, buf1, acc_v, sem0, sem1):
    w = lax.axis_index("s") * NC + lax.axis_index("c")
    base_row = w * RPW

    iota = lax.iota(jnp.int32, 16)
    zeros_f = jnp.zeros((16,), jnp.float32)
    bufs = (buf0, buf1)
    sems = (sem0, sem1)

    copies = []
    for g in range(2):
        copies.append(pltpu.async_copy(
            rows_in.at[pl.ds(base_row + g * (16 * F), 16 * F)],
            bufs[g], sems[g]))

    for g in range(GROUPS):
        s = g % 2
        buf = bufs[s]
        copies[g].wait()

        row_idx = [iota * F + f for f in range(F)]

        def _lane(d, acc):
            dvec = jnp.full((16,), d, jnp.int32)
            sa0 = sa1 = qa0 = qa1 = zeros_f
            for f in range(0, F, 2):
                e0 = plsc.load_gather(buf, [row_idx[f], dvec])
                e1 = plsc.load_gather(buf, [row_idx[f + 1], dvec])
                sa0 = sa0 + e0
                qa0 = qa0 + e0 * e0
                sa1 = sa1 + e1
                qa1 = qa1 + e1 * e1
            sa = sa0 + sa1
            return acc + (sa * sa - (qa0 + qa1))

        acc = lax.fori_loop(0, D, _lane, zeros_f)
        acc_v[pl.ds(g * 16, 16)] = 0.5 * acc

        if g + 2 < GROUPS:
            copies.append(pltpu.async_copy(
                rows_in.at[pl.ds(base_row + (g + 2) * (16 * F), 16 * F)],
                bufs[s], sems[s]))

    pltpu.sync_copy(acc_v, acc_out.at[pl.ds(w * BPW, BPW)])


@functools.partial(
    pl.kernel,
    mesh=_mesh,
    compiler_params=pltpu.CompilerParams(
        needs_layout_passes=False, use_tc_tiling_on_sc=False),
    out_type=jax.ShapeDtypeStruct((B,), jnp.float32),
    scratch_types=[
        pltpu.VMEM((NCHUNK, CHUNK), jnp.int32),
        pltpu.VMEM((RPW,), jnp.float32),
        pltpu.VMEM((BPW,), jnp.float32),
        pltpu.VMEM((BPW,), jnp.float32),
        pltpu.SemaphoreType.DMA,
        pltpu.SemaphoreType.DMA,
    ],
)
def _bias_kernel(idx_hbm, bias_hbm, acc_hbm, out1_hbm,
                 idx_v, bias_v, acc_v, out_v, bsem, asem):
    wid = lax.axis_index("s") * NC + lax.axis_index("c")
    base_row = wid * RPW
    base_samp = wid * BPW

    pltpu.sync_copy(idx_hbm.at[wid], idx_v)
    acc_cp = pltpu.async_copy(acc_hbm.at[pl.ds(base_samp, BPW)], acc_v, asem)

    iota = lax.iota(jnp.int32, 16)

    def _add_off(t, carry):
        j = t // 8
        col = (t - j * 8) * 16
        rvec = (base_row + t * 16) + iota
        fvec = lax.rem(rvec, F)
        idx_v[j, pl.ds(col, 16)] = idx_v[j, pl.ds(col, 16)] + fvec * V
        return carry

    lax.fori_loop(0, NCHUNK * 8, _add_off, 0)

    bcopies = []
    for j in range(NCHUNK):
        bcopies.append(pltpu.async_copy(
            bias_hbm.at[idx_v.at[j]], bias_v.at[pl.ds(j * CHUNK, CHUNK)],
            bsem))
    for cp in bcopies:
        cp.wait()
    acc_cp.wait()

    zeros_f = jnp.zeros((16,), jnp.float32)
    stride = iota * F

    def _group(g, carry):
        rb = g * (16 * F)
        bias_acc = zeros_f
        for f in range(F):
            bias_acc = bias_acc + plsc.load_gather(bias_v, [stride + rb + f])
        out_v[pl.ds(g * 16, 16)] = bias_acc + acc_v[pl.ds(g * 16, 16)]
        return carry

    lax.fori_loop(0, GROUPS, _group, 0)

    pltpu.sync_copy(out_v, out1_hbm.at[pl.ds(base_samp, BPW)])


def kernel(field_indices, emb_tables, bias_tables):
    idx3 = field_indices.T.reshape(F, 32, 128)
    embn = emb_tables.transpose(0, 2, 1).reshape(F * D // 8, 8, V)
    emb_tail = (emb_tables[:, V - TAIL:, :]
                .transpose(0, 2, 1).reshape(F * D // 8, 8, TAIL))
    lists, cnts = _bucket_kernel(idx3)
    rows = _gather_kernel(embn, emb_tail, lists, cnts)
    acc = _stats_kernel(rows)
    idx_w = field_indices.reshape(NW, NCHUNK, CHUNK)
    bias_flat = bias_tables.reshape(F * V)
    out1 = _bias_kernel(idx_w, bias_flat, acc)
    embeds = rows[:, :D].reshape(B, F, D)
    return (out1.reshape(B, 1), embeds)
